# baseline (speedup 1.0000x reference)
import functools

import jax
import jax.numpy as jnp
from jax import lax
from jax.experimental import pallas as pl
from jax.experimental.pallas import tpu as pltpu
from jax.experimental.pallas import tpu_sc as plsc

T = 2048
D = 2048
E = 8
F = 1408
K = 2
RB = 128
NBLK = 40
NROWS = NBLK * RB
NC = 2
NS = 16
NW = NC * NS
LANES = 16

_f32 = jnp.float32
_i32 = jnp.int32


def _router_body(x_ref, g_ref, s1_ref, s2_ref, p1_ref, p2_ref, be_ref):
    x = x_ref[...]
    g = g_ref[...]
    logits = jnp.dot(x, g, preferred_element_type=_f32)
    lane = lax.broadcasted_iota(_i32, logits.shape, 1)
    logits = jnp.where(lane < E, logits, -1e30)
    m = jnp.max(logits, axis=1, keepdims=True)
    p = jnp.exp(logits - m)
    p = jnp.where(lane < E, p, 0.0)
    s = p / jnp.sum(p, axis=1, keepdims=True)
    m1 = jnp.max(s, axis=1, keepdims=True)
    i1 = jnp.min(jnp.where(s >= m1, lane, 128), axis=1, keepdims=True)
    s_x = jnp.where(lane == i1, -1.0, s)
    m2 = jnp.max(s_x, axis=1, keepdims=True)
    i2 = jnp.min(jnp.where(s_x >= m2, lane, 128), axis=1, keepdims=True)
    s1_ref[...] = m1
    s2_ref[...] = m2

    bf = jnp.bfloat16
    r0 = lax.broadcasted_iota(_i32, (T, T), 0)
    c0 = lax.broadcasted_iota(_i32, (T, T), 1)
    tm = (r0 >= c0).astype(bf)
    m1h = (lane == i1).astype(bf)
    m2h = (lane == i2).astype(bf)
    cum1 = jnp.dot(tm, m1h, preferred_element_type=_f32)
    cum2 = jnp.dot(tm, m2h, preferred_element_type=_f32)
    cnt1 = cum1[T - 1:T, :]
    counts = (cnt1 + cum2[T - 1:T, :]).astype(_i32)
    padded = ((counts + (RB - 1)) >> 7) << 7
    ru = lax.broadcasted_iota(_i32, (128, 128), 0)
    cu = lax.broadcasted_iota(_i32, (128, 128), 1)
    sut = (ru < cu).astype(_f32)
    offs = jnp.dot(padded.astype(_f32), sut, preferred_element_type=_f32,
                   precision=lax.Precision.HIGHEST)
    m1f = m1h.astype(_f32)
    m2f = m2h.astype(_f32)
    rank1 = jnp.sum(m1f * cum1, axis=1, keepdims=True) - 1.0
    rank2 = (jnp.sum(m2f * (cum2 + cnt1), axis=1, keepdims=True) - 1.0)
    off1 = jnp.sum(m1f * offs, axis=1, keepdims=True)
    off2 = jnp.sum(m2f * offs, axis=1, keepdims=True)
    p1_ref[...] = (off1 + rank1).astype(_i32)
    p2_ref[...] = (off2 + rank2).astype(_i32)
    ends = offs + padded.astype(_f32)
    bpos = (lax.broadcasted_iota(_i32, (128, 128), 0) * RB).astype(_f32)
    ge = jnp.logical_and(bpos >= ends, lane[:128, :] < E).astype(_i32)
    be = jnp.sum(ge, axis=1, keepdims=True)
    be_ref[...] = jnp.minimum(be, E - 1)


def _router(xf, gate_padded):
    return pl.pallas_call(
        _router_body,
        out_shape=(
            jax.ShapeDtypeStruct((T, 1), _f32),
            jax.ShapeDtypeStruct((T, 1), _f32),
            jax.ShapeDtypeStruct((T, 1), _i32),
            jax.ShapeDtypeStruct((T, 1), _i32),
            jax.ShapeDtypeStruct((128, 1), _i32),
        ),
        compiler_params=pltpu.CompilerParams(
            vmem_limit_bytes=60 * 1024 * 1024),
    )(xf, gate_padded)


_HALF = NROWS
_DCH = 40
_DW = D // 2
_bf16 = jnp.bfloat16


def _make_dispatch_body(lo):
    def body(p1_hbm, p2_hbm, x_hbm, rin_hbm,
             posn_v, gidx_v, rows0_v, rows1_v,
             sg0, sg1, sw0, sw1):
        wid = lax.axis_index("s") * NC + lax.axis_index("c")
        iota = lax.iota(_i32, LANES)

        pltpu.sync_copy(p1_hbm, posn_v.at[pl.ds(0, T)])
        pltpu.sync_copy(p2_hbm, posn_v.at[pl.ds(T, T)])

        def bodyz(q, _):
            gidx_v[pl.ds(q * LANES, LANES)] = jnp.zeros((LANES,), _i32)
            return 0
        lax.fori_loop(0, NROWS // LANES, bodyz, 0)

        def body2(p, base):
            pos = posn_v[pl.ds(p * LANES, LANES)]
            tok = (iota + base) & (T - 1)
            plsc.store_scatter(gidx_v, [pos], tok)
            return base + LANES
        lax.fori_loop(0, (T * K) // LANES, body2,
                      jnp.zeros((LANES,), _i32))

        rows_per_w = _HALF // NW
        n = rows_per_w // _DCH
        base = wid * rows_per_w
        rows = (rows0_v, rows1_v)
        sg = (sg0, sg1)
        sw = (sw0, sw1)
        hg = {}
        hw = {}
        for c in range(n):
            p = c % 2
            if c >= 2:
                hw[c - 2].wait()
            hg[c] = pltpu.make_async_copy(
                x_hbm.at[gidx_v.at[pl.ds(lo + base + c * _DCH, _DCH)]],
                rows[p], sg[p])
            hg[c].start()
            if c >= 1:
                qq = (c - 1) % 2
                hg[c - 1].wait()
                hw[c - 1] = pltpu.make_async_copy(
                    rows[qq],
                    rin_hbm.at[pl.ds(base + (c - 1) * _DCH, _DCH)], sw[qq])
                hw[c - 1].start()
        hg[n - 1].wait()
        hw[n - 1] = pltpu.make_async_copy(
            rows[(n - 1) % 2],
            rin_hbm.at[pl.ds(base + (n - 1) * _DCH, _DCH)], sw[(n - 1) % 2])
        hw[n - 1].start()
        for c in (n - 2, n - 1):
            hw[c].wait()
    return body


def _dispatch_half(p1f, p2f, xb3, lo):
    mesh = plsc.VectorSubcoreMesh(core_axis_name="c", subcore_axis_name="s")
    fn = pl.kernel(
        _make_dispatch_body(lo),
        mesh=mesh,
        compiler_params=pltpu.CompilerParams(needs_layout_passes=False),
        out_type=jax.ShapeDtypeStruct((_HALF, _DW), _i32),
        scratch_types=[
            pltpu.VMEM((T * K,), _i32),
            pltpu.VMEM((NROWS,), _i32),
            pltpu.VMEM((_DCH, _DW), _i32),
            pltpu.VMEM((_DCH, _DW), _i32),
            pltpu.SemaphoreType.DMA,
            pltpu.SemaphoreType.DMA,
            pltpu.SemaphoreType.DMA,
            pltpu.SemaphoreType.DMA,
        ],
    )
    return fn(p1f, p2f, xb3)


def _ffn_body(bexp_ref, rin_ref, w1_ref, w3_ref, w2_ref, out_ref):
    r = rin_ref[...]
    w1 = w1_ref[0]
    w3 = w3_ref[0]
    w2 = w2_ref[0]
    a = jnp.dot(r, w1, preferred_element_type=_f32)
    b = jnp.dot(r, w3, preferred_element_type=_f32)
    h = a * (1.0 / (1.0 + jnp.exp(-a))) * b
    out_ref[...] = jnp.dot(h.astype(_bf16), w2,
                           preferred_element_type=_f32).astype(_bf16)


def _ffn_half(bexp, rin_half, w1, w3, w2, bbase):
    grid_spec = pltpu.PrefetchScalarGridSpec(
        num_scalar_prefetch=1,
        grid=(NBLK,),
        in_specs=[
            pl.BlockSpec((RB, D), lambda i, b: (i, 0)),
            pl.BlockSpec((1, D, F), lambda i, b: (b[bbase + i], 0, 0)),
            pl.BlockSpec((1, D, F), lambda i, b: (b[bbase + i], 0, 0)),
            pl.BlockSpec((1, F, D), lambda i, b: (b[bbase + i], 0, 0)),
        ],
        out_specs=pl.BlockSpec((RB, D), lambda i, b: (i, 0)),
    )
    return pl.pallas_call(
        _ffn_body,
        grid_spec=grid_spec,
        out_shape=jax.ShapeDtypeStruct((_HALF, D), _bf16),
        compiler_params=pltpu.CompilerParams(
            vmem_limit_bytes=62 * 1024 * 1024),
    )(bexp, rin_half, w1, w3, w2)


_GCH = 32


def _cgather_body(rout_hbm, p1_hbm, p2_hbm, o1_hbm, o2_hbm,
                  idx0_v, idx1_v, rows0_v, rows1_v, sg0, sg1, sw0, sw1):
    wid = lax.axis_index("s") * NC + lax.axis_index("c")
    per_w = T // NW
    nh = per_w // _GCH
    base = wid * per_w
    idx = (idx0_v, idx1_v)
    rows = (rows0_v, rows1_v)
    sg = (sg0, sg1)
    sw = (sw0, sw1)
    plan = [(p1_hbm, o1_hbm, base + c * _GCH) for c in range(nh)]
    plan += [(p2_hbm, o2_hbm, base + c * _GCH) for c in range(nh)]
    n = len(plan)
    hg = {}
    hw = {}
    for c, (p_hbm, o_hbm, cb) in enumerate(plan):
        p = c % 2
        if c >= 2:
            hw[c - 2].wait()
        pltpu.sync_copy(p_hbm.at[pl.ds(cb, _GCH)], idx[p])
        hg[c] = pltpu.make_async_copy(rout_hbm.at[idx[p]], rows[p], sg[p])
        hg[c].start()
        if c >= 1:
            q = (c - 1) % 2
            _, o_prev, cb_prev = plan[c - 1]
            hg[c - 1].wait()
            hw[c - 1] = pltpu.make_async_copy(
                rows[q], o_prev.at[pl.ds(cb_prev, _GCH)], sw[q])
            hw[c - 1].start()
    _, o_last, cb_last = plan[n - 1]
    hg[n - 1].wait()
    hw[n - 1] = pltpu.make_async_copy(
        rows[(n - 1) % 2], o_last.at[pl.ds(cb_last, _GCH)], sw[(n - 1) % 2])
    hw[n - 1].start()
    hw[n - 2].wait()
    hw[n - 1].wait()


def _cgather(rout3, p1f, p2f):
    mesh = plsc.VectorSubcoreMesh(core_axis_name="c", subcore_axis_name="s")
    fn = pl.kernel(
        _cgather_body,
        mesh=mesh,
        compiler_params=pltpu.CompilerParams(needs_layout_passes=False),
        out_type=(
            jax.ShapeDtypeStruct((T, _DW), _i32),
            jax.ShapeDtypeStruct((T, _DW), _i32),
        ),
        scratch_types=[
            pltpu.VMEM((_GCH,), _i32),
            pltpu.VMEM((_GCH,), _i32),
            pltpu.VMEM((_GCH, _DW), _i32),
            pltpu.VMEM((_GCH, _DW), _i32),
            pltpu.SemaphoreType.DMA,
            pltpu.SemaphoreType.DMA,
            pltpu.SemaphoreType.DMA,
            pltpu.SemaphoreType.DMA,
        ],
    )
    return fn(rout3, p1f, p2f)


def _sharedffn_body(xb_ref, sw1_ref, sw3_ref, sw2_ref, out_ref):
    x = xb_ref[...]
    a = jnp.dot(x, sw1_ref[...], preferred_element_type=_f32)
    b = jnp.dot(x, sw3_ref[...], preferred_element_type=_f32)
    h = a * (1.0 / (1.0 + jnp.exp(-a))) * b
    out_ref[...] = jnp.dot(h.astype(jnp.bfloat16), sw2_ref[...],
                           preferred_element_type=_f32)


def _sharedffn(xb, sw1, sw3, sw2):
    nb = 16
    rb = T // nb
    return pl.pallas_call(
        _sharedffn_body,
        grid=(nb,),
        in_specs=[
            pl.BlockSpec((rb, D), lambda i: (i, 0)),
            pl.BlockSpec((D, F), lambda i: (0, 0)),
            pl.BlockSpec((D, F), lambda i: (0, 0)),
            pl.BlockSpec((F, D), lambda i: (0, 0)),
        ],
        out_specs=pl.BlockSpec((rb, D), lambda i: (i, 0)),
        out_shape=jax.ShapeDtypeStruct((T, D), _f32),
        compiler_params=pltpu.CompilerParams(
            vmem_limit_bytes=62 * 1024 * 1024),
    )(xb, sw1, sw3, sw2)


def _combine_body(sh_ref, o1_ref, o2_ref, s1_ref, s2_ref, out_ref):
    out_ref[...] = (sh_ref[...] + s1_ref[...] * o1_ref[...].astype(_f32)
                    + s2_ref[...] * o2_ref[...].astype(_f32))


def _combine(sh, o1, o2, s1, s2):
    nb = 16
    rb = T // nb
    return pl.pallas_call(
        _combine_body,
        grid=(nb,),
        in_specs=[
            pl.BlockSpec((rb, D), lambda i: (i, 0)),
            pl.BlockSpec((rb, D), lambda i: (i, 0)),
            pl.BlockSpec((rb, D), lambda i: (i, 0)),
            pl.BlockSpec((rb, 1), lambda i: (i, 0)),
            pl.BlockSpec((rb, 1), lambda i: (i, 0)),
        ],
        out_specs=pl.BlockSpec((rb, D), lambda i: (i, 0)),
        out_shape=jax.ShapeDtypeStruct((T, D), _f32),
    )(sh, o1, o2, s1, s2)


def kernel(x, gate, w1, w2, w3, sw1, sw2, sw3):
    xf = x.reshape(T, D)
    gate_padded = jnp.pad(gate, ((0, 0), (0, 128 - E)))
    s1, s2, pos1, pos2, bexp = _router(xf, gate_padded)
    p1f = pos1.reshape(T)
    p2f = pos2.reshape(T)
    xb = xf.astype(_bf16)

    def pack(a):
        return lax.bitcast_convert_type(a.reshape(a.shape[0], _DW, 2),
                                        _i32)

    def unpack(a):
        return lax.bitcast_convert_type(a, _bf16).reshape(a.shape[0], D)

    xw = pack(xb)
    bexp128 = bexp.reshape(128)
    w1b = w1.astype(_bf16)
    w3b = w3.astype(_bf16)
    w2b = w2.astype(_bf16)
    rin = _dispatch_half(p1f, p2f, xw, 0)
    rout = _ffn_half(bexp128, unpack(rin), w1b, w3b, w2b, 0)
    rout3 = pack(rout)
    o1, o2 = _cgather(rout3, p1f, p2f)
    sh = _sharedffn(xb, sw1.astype(_bf16), sw3.astype(_bf16),
                    sw2.astype(_bf16))
    out = _combine(sh, unpack(o1), unpack(o2), s1, s2)
    return out.reshape(1, T, D)


# speedup vs baseline: 2.3968x; 2.3968x over previous
import functools

import jax
import jax.numpy as jnp
from jax import lax
from jax.experimental import pallas as pl
from jax.experimental.pallas import tpu as pltpu
from jax.experimental.pallas import tpu_sc as plsc

T = 2048
D = 2048
E = 8
F = 1408
K = 2
RB = 128
NBLK = 40
NROWS = NBLK * RB
NC = 2
NS = 16
NW = NC * NS
LANES = 16

_f32 = jnp.float32
_i32 = jnp.int32
_u32 = jnp.uint32
_u16 = jnp.uint16
_DWORDS = D // 2


def _pack_cols(xb):
    lo = lax.bitcast_convert_type(xb[:, :_DWORDS], _u16).astype(_u32)
    hi = lax.bitcast_convert_type(xb[:, _DWORDS:], _u16).astype(_u32)
    return lax.bitcast_convert_type((hi << 16) | lo, _i32)


def _unpack_cols(w):
    wu = lax.bitcast_convert_type(w, _u32)
    lo = lax.bitcast_convert_type((wu & 0xFFFF).astype(_u16), jnp.bfloat16)
    hi = lax.bitcast_convert_type((wu >> 16).astype(_u16), jnp.bfloat16)
    return jnp.concatenate([lo, hi], axis=1)


def _router_body(x_ref, g_ref, s1_ref, s2_ref, p1_ref, p2_ref, be_ref,
                 xw_ref):
    x = x_ref[...]
    xw_ref[...] = _pack_cols(x.astype(jnp.bfloat16))
    g = g_ref[...]
    logits = jnp.dot(x, g, preferred_element_type=_f32)
    lane = lax.broadcasted_iota(_i32, logits.shape, 1)
    logits = jnp.where(lane < E, logits, -1e30)
    m = jnp.max(logits, axis=1, keepdims=True)
    p = jnp.exp(logits - m)
    p = jnp.where(lane < E, p, 0.0)
    s = p / jnp.sum(p, axis=1, keepdims=True)
    m1 = jnp.max(s, axis=1, keepdims=True)
    i1 = jnp.min(jnp.where(s >= m1, lane, 128), axis=1, keepdims=True)
    s_x = jnp.where(lane == i1, -1.0, s)
    m2 = jnp.max(s_x, axis=1, keepdims=True)
    i2 = jnp.min(jnp.where(s_x >= m2, lane, 128), axis=1, keepdims=True)
    s1_ref[...] = m1
    s2_ref[...] = m2

    bf = jnp.bfloat16
    r0 = lax.broadcasted_iota(_i32, (T, T), 0)
    c0 = lax.broadcasted_iota(_i32, (T, T), 1)
    tm = (r0 >= c0).astype(bf)
    m1h = (lane == i1).astype(bf)
    m2h = (lane == i2).astype(bf)
    cum1 = jnp.dot(tm, m1h, preferred_element_type=_f32)
    cum2 = jnp.dot(tm, m2h, preferred_element_type=_f32)
    cnt1 = cum1[T - 1:T, :]
    counts = (cnt1 + cum2[T - 1:T, :]).astype(_i32)
    padded = ((counts + (RB - 1)) >> 7) << 7
    ru = lax.broadcasted_iota(_i32, (128, 128), 0)
    cu = lax.broadcasted_iota(_i32, (128, 128), 1)
    sut = (ru < cu).astype(_f32)
    offs = jnp.dot(padded.astype(_f32), sut, preferred_element_type=_f32,
                   precision=lax.Precision.HIGHEST)
    m1f = m1h.astype(_f32)
    m2f = m2h.astype(_f32)
    rank1 = jnp.sum(m1f * cum1, axis=1, keepdims=True) - 1.0
    rank2 = (jnp.sum(m2f * (cum2 + cnt1), axis=1, keepdims=True) - 1.0)
    off1 = jnp.sum(m1f * offs, axis=1, keepdims=True)
    off2 = jnp.sum(m2f * offs, axis=1, keepdims=True)
    p1_ref[...] = (off1 + rank1).astype(_i32)
    p2_ref[...] = (off2 + rank2).astype(_i32)
    ends = offs + padded.astype(_f32)
    bpos = (lax.broadcasted_iota(_i32, (128, 128), 0) * RB).astype(_f32)
    ge = jnp.logical_and(bpos >= ends, lane[:128, :] < E).astype(_i32)
    be = jnp.sum(ge, axis=1, keepdims=True)
    be_ref[...] = jnp.minimum(be, E - 1)


def _router(xf, gate_padded):
    return pl.pallas_call(
        _router_body,
        out_shape=(
            jax.ShapeDtypeStruct((T, 1), _f32),
            jax.ShapeDtypeStruct((T, 1), _f32),
            jax.ShapeDtypeStruct((T, 1), _i32),
            jax.ShapeDtypeStruct((T, 1), _i32),
            jax.ShapeDtypeStruct((128, 1), _i32),
            jax.ShapeDtypeStruct((T, _DWORDS), _i32),
        ),
        compiler_params=pltpu.CompilerParams(
            vmem_limit_bytes=60 * 1024 * 1024),
    )(xf, gate_padded)


_HALF = NROWS
_DCH = 40
_DW = D // 2
_bf16 = jnp.bfloat16


def _make_dispatch_body(lo):
    def body(p1_hbm, p2_hbm, x_hbm, rin_hbm,
             posn_v, gidx_v, rows0_v, rows1_v,
             sg0, sg1, sw0, sw1):
        wid = lax.axis_index("s") * NC + lax.axis_index("c")
        iota = lax.iota(_i32, LANES)

        pltpu.sync_copy(p1_hbm, posn_v.at[pl.ds(0, T)])
        pltpu.sync_copy(p2_hbm, posn_v.at[pl.ds(T, T)])

        def bodyz(q, _):
            gidx_v[pl.ds(q * LANES, LANES)] = jnp.zeros((LANES,), _i32)
            return 0
        lax.fori_loop(0, NROWS // LANES, bodyz, 0)

        def body2(p, base):
            pos = posn_v[pl.ds(p * LANES, LANES)]
            tok = (iota + base) & (T - 1)
            plsc.store_scatter(gidx_v, [pos], tok)
            return base + LANES
        lax.fori_loop(0, (T * K) // LANES, body2,
                      jnp.zeros((LANES,), _i32))

        rows_per_w = _HALF // NW
        n = rows_per_w // _DCH
        base = wid * rows_per_w
        rows = (rows0_v, rows1_v)
        sg = (sg0, sg1)
        sw = (sw0, sw1)
        hg = {}
        hw = {}
        for c in range(n):
            p = c % 2
            if c >= 2:
                hw[c - 2].wait()
            hg[c] = pltpu.make_async_copy(
                x_hbm.at[gidx_v.at[pl.ds(lo + base + c * _DCH, _DCH)]],
                rows[p], sg[p])
            hg[c].start()
            if c >= 1:
                qq = (c - 1) % 2
                hg[c - 1].wait()
                hw[c - 1] = pltpu.make_async_copy(
                    rows[qq],
                    rin_hbm.at[pl.ds(base + (c - 1) * _DCH, _DCH)], sw[qq])
                hw[c - 1].start()
        hg[n - 1].wait()
        hw[n - 1] = pltpu.make_async_copy(
            rows[(n - 1) % 2],
            rin_hbm.at[pl.ds(base + (n - 1) * _DCH, _DCH)], sw[(n - 1) % 2])
        hw[n - 1].start()
        for c in (n - 2, n - 1):
            hw[c].wait()
    return body


def _dispatch_half(p1f, p2f, xb3, lo):
    mesh = plsc.VectorSubcoreMesh(core_axis_name="c", subcore_axis_name="s")
    fn = pl.kernel(
        _make_dispatch_body(lo),
        mesh=mesh,
        compiler_params=pltpu.CompilerParams(needs_layout_passes=False),
        out_type=jax.ShapeDtypeStruct((_HALF, _DW), _i32),
        scratch_types=[
            pltpu.VMEM((T * K,), _i32),
            pltpu.VMEM((NROWS,), _i32),
            pltpu.VMEM((_DCH, _DW), _i32),
            pltpu.VMEM((_DCH, _DW), _i32),
            pltpu.SemaphoreType.DMA,
            pltpu.SemaphoreType.DMA,
            pltpu.SemaphoreType.DMA,
            pltpu.SemaphoreType.DMA,
        ],
    )
    return fn(p1f, p2f, xb3)


def _ffn_body(bexp_ref, rin_ref, w1_ref, w3_ref, w2_ref, out_ref):
    r = _unpack_cols(rin_ref[...])
    w1 = w1_ref[0]
    w3 = w3_ref[0]
    w2 = w2_ref[0]
    a = jnp.dot(r, w1, preferred_element_type=_f32)
    b = jnp.dot(r, w3, preferred_element_type=_f32)
    h = a * (1.0 / (1.0 + jnp.exp(-a))) * b
    o = jnp.dot(h.astype(_bf16), w2, preferred_element_type=_f32)
    out_ref[...] = _pack_cols(o.astype(_bf16))


def _ffn_half(bexp, rin_half, w1, w3, w2, bbase):
    grid_spec = pltpu.PrefetchScalarGridSpec(
        num_scalar_prefetch=1,
        grid=(NBLK,),
        in_specs=[
            pl.BlockSpec((RB, _DWORDS), lambda i, b: (i, 0)),
            pl.BlockSpec((1, D, F), lambda i, b: (b[bbase + i], 0, 0)),
            pl.BlockSpec((1, D, F), lambda i, b: (b[bbase + i], 0, 0)),
            pl.BlockSpec((1, F, D), lambda i, b: (b[bbase + i], 0, 0)),
        ],
        out_specs=pl.BlockSpec((RB, _DWORDS), lambda i, b: (i, 0)),
    )
    return pl.pallas_call(
        _ffn_body,
        grid_spec=grid_spec,
        out_shape=jax.ShapeDtypeStruct((_HALF, _DWORDS), _i32),
        compiler_params=pltpu.CompilerParams(
            vmem_limit_bytes=62 * 1024 * 1024),
    )(bexp, rin_half, w1, w3, w2)


_GCH = 32


def _cgather_body(rout_hbm, p1_hbm, p2_hbm, o1_hbm, o2_hbm,
                  idx0_v, idx1_v, rows0_v, rows1_v, sg0, sg1, sw0, sw1):
    wid = lax.axis_index("s") * NC + lax.axis_index("c")
    per_w = T // NW
    nh = per_w // _GCH
    base = wid * per_w
    idx = (idx0_v, idx1_v)
    rows = (rows0_v, rows1_v)
    sg = (sg0, sg1)
    sw = (sw0, sw1)
    plan = [(p1_hbm, o1_hbm, base + c * _GCH) for c in range(nh)]
    plan += [(p2_hbm, o2_hbm, base + c * _GCH) for c in range(nh)]
    n = len(plan)
    hg = {}
    hw = {}
    for c, (p_hbm, o_hbm, cb) in enumerate(plan):
        p = c % 2
        if c >= 2:
            hw[c - 2].wait()
        pltpu.sync_copy(p_hbm.at[pl.ds(cb, _GCH)], idx[p])
        hg[c] = pltpu.make_async_copy(rout_hbm.at[idx[p]], rows[p], sg[p])
        hg[c].start()
        if c >= 1:
            q = (c - 1) % 2
            _, o_prev, cb_prev = plan[c - 1]
            hg[c - 1].wait()
            hw[c - 1] = pltpu.make_async_copy(
                rows[q], o_prev.at[pl.ds(cb_prev, _GCH)], sw[q])
            hw[c - 1].start()
    _, o_last, cb_last = plan[n - 1]
    hg[n - 1].wait()
    hw[n - 1] = pltpu.make_async_copy(
        rows[(n - 1) % 2], o_last.at[pl.ds(cb_last, _GCH)], sw[(n - 1) % 2])
    hw[n - 1].start()
    hw[n - 2].wait()
    hw[n - 1].wait()


def _cgather(rout3, p1f, p2f):
    mesh = plsc.VectorSubcoreMesh(core_axis_name="c", subcore_axis_name="s")
    fn = pl.kernel(
        _cgather_body,
        mesh=mesh,
        compiler_params=pltpu.CompilerParams(needs_layout_passes=False),
        out_type=(
            jax.ShapeDtypeStruct((T, _DW), _i32),
            jax.ShapeDtypeStruct((T, _DW), _i32),
        ),
        scratch_types=[
            pltpu.VMEM((_GCH,), _i32),
            pltpu.VMEM((_GCH,), _i32),
            pltpu.VMEM((_GCH, _DW), _i32),
            pltpu.VMEM((_GCH, _DW), _i32),
            pltpu.SemaphoreType.DMA,
            pltpu.SemaphoreType.DMA,
            pltpu.SemaphoreType.DMA,
            pltpu.SemaphoreType.DMA,
        ],
    )
    return fn(rout3, p1f, p2f)


def _sharedffn_body(xb_ref, sw1_ref, sw3_ref, sw2_ref, out_ref):
    x = xb_ref[...]
    a = jnp.dot(x, sw1_ref[...], preferred_element_type=_f32)
    b = jnp.dot(x, sw3_ref[...], preferred_element_type=_f32)
    h = a * (1.0 / (1.0 + jnp.exp(-a))) * b
    out_ref[...] = jnp.dot(h.astype(jnp.bfloat16), sw2_ref[...],
                           preferred_element_type=_f32)


def _sharedffn(xb, sw1, sw3, sw2):
    nb = 16
    rb = T // nb
    return pl.pallas_call(
        _sharedffn_body,
        grid=(nb,),
        in_specs=[
            pl.BlockSpec((rb, D), lambda i: (i, 0)),
            pl.BlockSpec((D, F), lambda i: (0, 0)),
            pl.BlockSpec((D, F), lambda i: (0, 0)),
            pl.BlockSpec((F, D), lambda i: (0, 0)),
        ],
        out_specs=pl.BlockSpec((rb, D), lambda i: (i, 0)),
        out_shape=jax.ShapeDtypeStruct((T, D), _f32),
        compiler_params=pltpu.CompilerParams(
            vmem_limit_bytes=62 * 1024 * 1024),
    )(xb, sw1, sw3, sw2)


def _combine_body(sh_ref, o1_ref, o2_ref, s1_ref, s2_ref, out_ref):
    o1 = _unpack_cols(o1_ref[...]).astype(_f32)
    o2 = _unpack_cols(o2_ref[...]).astype(_f32)
    out_ref[...] = (sh_ref[...] + s1_ref[...] * o1 + s2_ref[...] * o2)


def _combine(sh, o1, o2, s1, s2):
    nb = 16
    rb = T // nb
    return pl.pallas_call(
        _combine_body,
        grid=(nb,),
        in_specs=[
            pl.BlockSpec((rb, D), lambda i: (i, 0)),
            pl.BlockSpec((rb, _DWORDS), lambda i: (i, 0)),
            pl.BlockSpec((rb, _DWORDS), lambda i: (i, 0)),
            pl.BlockSpec((rb, 1), lambda i: (i, 0)),
            pl.BlockSpec((rb, 1), lambda i: (i, 0)),
        ],
        out_specs=pl.BlockSpec((rb, D), lambda i: (i, 0)),
        out_shape=jax.ShapeDtypeStruct((T, D), _f32),
    )(sh, o1, o2, s1, s2)


def kernel(x, gate, w1, w2, w3, sw1, sw2, sw3):
    xf = x.reshape(T, D)
    gate_padded = jnp.pad(gate, ((0, 0), (0, 128 - E)))
    s1, s2, pos1, pos2, bexp, xw = _router(xf, gate_padded)
    p1f = pos1.reshape(T)
    p2f = pos2.reshape(T)
    bexp128 = bexp.reshape(128)
    w1b = w1.astype(_bf16)
    w3b = w3.astype(_bf16)
    w2b = w2.astype(_bf16)
    rin = _dispatch_half(p1f, p2f, xw, 0)
    rout = _ffn_half(bexp128, rin, w1b, w3b, w2b, 0)
    o1, o2 = _cgather(rout, p1f, p2f)
    sh = _sharedffn(xf.astype(_bf16), sw1.astype(_bf16), sw3.astype(_bf16),
                    sw2.astype(_bf16))
    out = _combine(sh, o1, o2, s1, s2)
    return out.reshape(1, T, D)


# speedup vs baseline: 2.4859x; 1.0372x over previous
import functools

import jax
import jax.numpy as jnp
from jax import lax
from jax.experimental import pallas as pl
from jax.experimental.pallas import tpu as pltpu
from jax.experimental.pallas import tpu_sc as plsc

T = 2048
D = 2048
E = 8
F = 1408
K = 2
RB = 128
NBLK = 40
NROWS = NBLK * RB
NC = 2
NS = 16
NW = NC * NS
LANES = 16

_f32 = jnp.float32
_i32 = jnp.int32
_u32 = jnp.uint32
_u16 = jnp.uint16
_DWORDS = D // 2


def _pack_cols(xb):
    lo = lax.bitcast_convert_type(xb[:, :_DWORDS], _u16).astype(_u32)
    hi = lax.bitcast_convert_type(xb[:, _DWORDS:], _u16).astype(_u32)
    return lax.bitcast_convert_type((hi << 16) | lo, _i32)


def _unpack_cols(w):
    wu = lax.bitcast_convert_type(w, _u32)
    lo = lax.bitcast_convert_type((wu & 0xFFFF).astype(_u16), jnp.bfloat16)
    hi = lax.bitcast_convert_type((wu >> 16).astype(_u16), jnp.bfloat16)
    return jnp.concatenate([lo, hi], axis=1)


def _router_body(x_ref, g_ref, s1_ref, s2_ref, p1_ref, p2_ref, be_ref,
                 xw_ref):
    x = x_ref[...]
    xw_ref[...] = _pack_cols(x.astype(jnp.bfloat16))
    g = g_ref[...]
    logits = jnp.dot(x, g, preferred_element_type=_f32)
    lane = lax.broadcasted_iota(_i32, logits.shape, 1)
    logits = jnp.where(lane < E, logits, -1e30)
    m = jnp.max(logits, axis=1, keepdims=True)
    p = jnp.exp(logits - m)
    p = jnp.where(lane < E, p, 0.0)
    s = p / jnp.sum(p, axis=1, keepdims=True)
    m1 = jnp.max(s, axis=1, keepdims=True)
    i1 = jnp.min(jnp.where(s >= m1, lane, 128), axis=1, keepdims=True)
    s_x = jnp.where(lane == i1, -1.0, s)
    m2 = jnp.max(s_x, axis=1, keepdims=True)
    i2 = jnp.min(jnp.where(s_x >= m2, lane, 128), axis=1, keepdims=True)
    s1_ref[...] = m1
    s2_ref[...] = m2

    bf = jnp.bfloat16
    r0 = lax.broadcasted_iota(_i32, (T, T), 0)
    c0 = lax.broadcasted_iota(_i32, (T, T), 1)
    tm = (r0 >= c0).astype(bf)
    m1h = (lane == i1).astype(bf)
    m2h = (lane == i2).astype(bf)
    cum1 = jnp.dot(tm, m1h, preferred_element_type=_f32)
    cum2 = jnp.dot(tm, m2h, preferred_element_type=_f32)
    cnt1 = cum1[T - 1:T, :]
    counts = (cnt1 + cum2[T - 1:T, :]).astype(_i32)
    padded = ((counts + (RB - 1)) >> 7) << 7
    ru = lax.broadcasted_iota(_i32, (128, 128), 0)
    cu = lax.broadcasted_iota(_i32, (128, 128), 1)
    sut = (ru < cu).astype(_f32)
    offs = jnp.dot(padded.astype(_f32), sut, preferred_element_type=_f32,
                   precision=lax.Precision.HIGHEST)
    m1f = m1h.astype(_f32)
    m2f = m2h.astype(_f32)
    rank1 = jnp.sum(m1f * cum1, axis=1, keepdims=True) - 1.0
    rank2 = (jnp.sum(m2f * (cum2 + cnt1), axis=1, keepdims=True) - 1.0)
    off1 = jnp.sum(m1f * offs, axis=1, keepdims=True)
    off2 = jnp.sum(m2f * offs, axis=1, keepdims=True)
    p1_ref[...] = (off1 + rank1).astype(_i32)
    p2_ref[...] = (off2 + rank2).astype(_i32)
    ends = offs + padded.astype(_f32)
    bpos = (lax.broadcasted_iota(_i32, (128, 128), 0) * RB).astype(_f32)
    ge = jnp.logical_and(bpos >= ends, lane[:128, :] < E).astype(_i32)
    be = jnp.sum(ge, axis=1, keepdims=True)
    be_ref[...] = jnp.minimum(be, E - 1)


def _router(xf, gate_padded):
    return pl.pallas_call(
        _router_body,
        out_shape=(
            jax.ShapeDtypeStruct((T, 1), _f32),
            jax.ShapeDtypeStruct((T, 1), _f32),
            jax.ShapeDtypeStruct((T, 1), _i32),
            jax.ShapeDtypeStruct((T, 1), _i32),
            jax.ShapeDtypeStruct((128, 1), _i32),
            jax.ShapeDtypeStruct((T, _DWORDS), _i32),
        ),
        compiler_params=pltpu.CompilerParams(
            vmem_limit_bytes=60 * 1024 * 1024),
    )(xf, gate_padded)


_HALF = NROWS
_DCH = 40
_DW = D // 2
_bf16 = jnp.bfloat16


def _make_dispatch_body(lo):
    def body(p1_hbm, p2_hbm, x_hbm, rin_hbm,
             posn_v, gidx_v, rows0_v, rows1_v,
             sg0, sg1, sw0, sw1):
        wid = lax.axis_index("s") * NC + lax.axis_index("c")
        iota = lax.iota(_i32, LANES)

        pltpu.sync_copy(p1_hbm, posn_v.at[pl.ds(0, T)])
        pltpu.sync_copy(p2_hbm, posn_v.at[pl.ds(T, T)])

        def bodyz(q, _):
            gidx_v[pl.ds(q * LANES, LANES)] = jnp.zeros((LANES,), _i32)
            return 0
        lax.fori_loop(0, NROWS // LANES, bodyz, 0)

        def body2(p, base):
            pos = posn_v[pl.ds(p * LANES, LANES)]
            tok = (iota + base) & (T - 1)
            plsc.store_scatter(gidx_v, [pos], tok)
            return base + LANES
        lax.fori_loop(0, (T * K) // LANES, body2,
                      jnp.zeros((LANES,), _i32))

        rows_per_w = _HALF // NW
        n = rows_per_w // _DCH
        base = wid * rows_per_w
        rows = (rows0_v, rows1_v)
        sg = (sg0, sg1)
        sw = (sw0, sw1)
        hg = {}
        hw = {}
        for c in range(n):
            p = c % 2
            if c >= 2:
                hw[c - 2].wait()
            hg[c] = pltpu.make_async_copy(
                x_hbm.at[gidx_v.at[pl.ds(lo + base + c * _DCH, _DCH)]],
                rows[p], sg[p])
            hg[c].start()
            if c >= 1:
                qq = (c - 1) % 2
                hg[c - 1].wait()
                hw[c - 1] = pltpu.make_async_copy(
                    rows[qq],
                    rin_hbm.at[pl.ds(base + (c - 1) * _DCH, _DCH)], sw[qq])
                hw[c - 1].start()
        hg[n - 1].wait()
        hw[n - 1] = pltpu.make_async_copy(
            rows[(n - 1) % 2],
            rin_hbm.at[pl.ds(base + (n - 1) * _DCH, _DCH)], sw[(n - 1) % 2])
        hw[n - 1].start()
        for c in (n - 2, n - 1):
            hw[c].wait()
    return body


def _dispatch_half(p1f, p2f, xb3, lo):
    mesh = plsc.VectorSubcoreMesh(core_axis_name="c", subcore_axis_name="s")
    fn = pl.kernel(
        _make_dispatch_body(lo),
        mesh=mesh,
        compiler_params=pltpu.CompilerParams(needs_layout_passes=False),
        out_type=jax.ShapeDtypeStruct((_HALF, _DW), _i32),
        scratch_types=[
            pltpu.VMEM((T * K,), _i32),
            pltpu.VMEM((NROWS,), _i32),
            pltpu.VMEM((_DCH, _DW), _i32),
            pltpu.VMEM((_DCH, _DW), _i32),
            pltpu.SemaphoreType.DMA,
            pltpu.SemaphoreType.DMA,
            pltpu.SemaphoreType.DMA,
            pltpu.SemaphoreType.DMA,
        ],
    )
    return fn(p1f, p2f, xb3)


def _ffn_body(bexp_ref, rin_ref, w1_ref, w3_ref, w2_ref, out_ref):
    r = _unpack_cols(rin_ref[...])
    w1 = w1_ref[0]
    w3 = w3_ref[0]
    w2 = w2_ref[0]
    a = jnp.dot(r, w1, preferred_element_type=_f32)
    b = jnp.dot(r, w3, preferred_element_type=_f32)
    h = a * (1.0 / (1.0 + jnp.exp(-a))) * b
    o = jnp.dot(h.astype(_bf16), w2, preferred_element_type=_f32)
    out_ref[...] = _pack_cols(o.astype(_bf16))


def _ffn_half(bexp, rin_half, w1, w3, w2, bbase):
    grid_spec = pltpu.PrefetchScalarGridSpec(
        num_scalar_prefetch=1,
        grid=(NBLK,),
        in_specs=[
            pl.BlockSpec((RB, _DWORDS), lambda i, b: (i, 0)),
            pl.BlockSpec((1, D, F), lambda i, b: (b[bbase + i], 0, 0)),
            pl.BlockSpec((1, D, F), lambda i, b: (b[bbase + i], 0, 0)),
            pl.BlockSpec((1, F, D), lambda i, b: (b[bbase + i], 0, 0)),
        ],
        out_specs=pl.BlockSpec((RB, _DWORDS), lambda i, b: (i, 0)),
    )
    return pl.pallas_call(
        _ffn_body,
        grid_spec=grid_spec,
        out_shape=jax.ShapeDtypeStruct((_HALF, _DWORDS), _i32),
        compiler_params=pltpu.CompilerParams(
            vmem_limit_bytes=62 * 1024 * 1024),
    )(bexp, rin_half, w1, w3, w2)


_GCH = 32


def _cgather_body(rout_hbm, p1_hbm, p2_hbm, o1_hbm, o2_hbm,
                  idx0_v, idx1_v, rows0_v, rows1_v, sg0, sg1, sw0, sw1):
    wid = lax.axis_index("s") * NC + lax.axis_index("c")
    per_w = T // NW
    nh = per_w // _GCH
    base = wid * per_w
    idx = (idx0_v, idx1_v)
    rows = (rows0_v, rows1_v)
    sg = (sg0, sg1)
    sw = (sw0, sw1)
    plan = [(p1_hbm, o1_hbm, base + c * _GCH) for c in range(nh)]
    plan += [(p2_hbm, o2_hbm, base + c * _GCH) for c in range(nh)]
    n = len(plan)
    hg = {}
    hw = {}
    for c, (p_hbm, o_hbm, cb) in enumerate(plan):
        p = c % 2
        if c >= 2:
            hw[c - 2].wait()
        pltpu.sync_copy(p_hbm.at[pl.ds(cb, _GCH)], idx[p])
        hg[c] = pltpu.make_async_copy(rout_hbm.at[idx[p]], rows[p], sg[p])
        hg[c].start()
        if c >= 1:
            q = (c - 1) % 2
            _, o_prev, cb_prev = plan[c - 1]
            hg[c - 1].wait()
            hw[c - 1] = pltpu.make_async_copy(
                rows[q], o_prev.at[pl.ds(cb_prev, _GCH)], sw[q])
            hw[c - 1].start()
    _, o_last, cb_last = plan[n - 1]
    hg[n - 1].wait()
    hw[n - 1] = pltpu.make_async_copy(
        rows[(n - 1) % 2], o_last.at[pl.ds(cb_last, _GCH)], sw[(n - 1) % 2])
    hw[n - 1].start()
    hw[n - 2].wait()
    hw[n - 1].wait()


def _cgather(rout3, p1f, p2f):
    mesh = plsc.VectorSubcoreMesh(core_axis_name="c", subcore_axis_name="s")
    fn = pl.kernel(
        _cgather_body,
        mesh=mesh,
        compiler_params=pltpu.CompilerParams(needs_layout_passes=False),
        out_type=(
            jax.ShapeDtypeStruct((T, _DW), _i32),
            jax.ShapeDtypeStruct((T, _DW), _i32),
        ),
        scratch_types=[
            pltpu.VMEM((_GCH,), _i32),
            pltpu.VMEM((_GCH,), _i32),
            pltpu.VMEM((_GCH, _DW), _i32),
            pltpu.VMEM((_GCH, _DW), _i32),
            pltpu.SemaphoreType.DMA,
            pltpu.SemaphoreType.DMA,
            pltpu.SemaphoreType.DMA,
            pltpu.SemaphoreType.DMA,
        ],
    )
    return fn(rout3, p1f, p2f)


def _sharedffn_body(xb_ref, sw1_ref, sw3_ref, sw2_ref, out_ref):
    x = xb_ref[...]
    a = jnp.dot(x, sw1_ref[...], preferred_element_type=_f32)
    b = jnp.dot(x, sw3_ref[...], preferred_element_type=_f32)
    h = a * (1.0 / (1.0 + jnp.exp(-a))) * b
    out_ref[...] = jnp.dot(h.astype(jnp.bfloat16), sw2_ref[...],
                           preferred_element_type=_f32)


def _sharedffn(xb, sw1, sw3, sw2):
    nb = 16
    rb = T // nb
    return pl.pallas_call(
        _sharedffn_body,
        grid=(nb,),
        in_specs=[
            pl.BlockSpec((rb, D), lambda i: (i, 0)),
            pl.BlockSpec((D, F), lambda i: (0, 0)),
            pl.BlockSpec((D, F), lambda i: (0, 0)),
            pl.BlockSpec((F, D), lambda i: (0, 0)),
        ],
        out_specs=pl.BlockSpec((rb, D), lambda i: (i, 0)),
        out_shape=jax.ShapeDtypeStruct((T, D), _f32),
        compiler_params=pltpu.CompilerParams(
            vmem_limit_bytes=62 * 1024 * 1024),
    )(xb, sw1, sw3, sw2)


def _combine_body(xb_ref, sw1_ref, sw3_ref, sw2_ref, o1_ref, o2_ref,
                  s1_ref, s2_ref, out_ref):
    x = xb_ref[...]
    a = jnp.dot(x, sw1_ref[...], preferred_element_type=_f32)
    b = jnp.dot(x, sw3_ref[...], preferred_element_type=_f32)
    h = a * (1.0 / (1.0 + jnp.exp(-a))) * b
    sh = jnp.dot(h.astype(_bf16), sw2_ref[...], preferred_element_type=_f32)
    o1 = _unpack_cols(o1_ref[...]).astype(_f32)
    o2 = _unpack_cols(o2_ref[...]).astype(_f32)
    out_ref[...] = sh + s1_ref[...] * o1 + s2_ref[...] * o2


def _combine(xb, sw1, sw3, sw2, o1, o2, s1, s2):
    nb = 16
    rb = T // nb
    return pl.pallas_call(
        _combine_body,
        grid=(nb,),
        in_specs=[
            pl.BlockSpec((rb, D), lambda i: (i, 0)),
            pl.BlockSpec((D, F), lambda i: (0, 0)),
            pl.BlockSpec((D, F), lambda i: (0, 0)),
            pl.BlockSpec((F, D), lambda i: (0, 0)),
            pl.BlockSpec((rb, _DWORDS), lambda i: (i, 0)),
            pl.BlockSpec((rb, _DWORDS), lambda i: (i, 0)),
            pl.BlockSpec((rb, 1), lambda i: (i, 0)),
            pl.BlockSpec((rb, 1), lambda i: (i, 0)),
        ],
        out_specs=pl.BlockSpec((rb, D), lambda i: (i, 0)),
        out_shape=jax.ShapeDtypeStruct((T, D), _f32),
        compiler_params=pltpu.CompilerParams(
            vmem_limit_bytes=62 * 1024 * 1024),
    )(xb, sw1, sw3, sw2, o1, o2, s1, s2)


def kernel(x, gate, w1, w2, w3, sw1, sw2, sw3):
    xf = x.reshape(T, D)
    gate_padded = jnp.pad(gate, ((0, 0), (0, 128 - E)))
    s1, s2, pos1, pos2, bexp, xw = _router(xf, gate_padded)
    p1f = pos1.reshape(T)
    p2f = pos2.reshape(T)
    bexp128 = bexp.reshape(128)
    w1b = w1.astype(_bf16)
    w3b = w3.astype(_bf16)
    w2b = w2.astype(_bf16)
    rin = _dispatch_half(p1f, p2f, xw, 0)
    rout = _ffn_half(bexp128, rin, w1b, w3b, w2b, 0)
    o1, o2 = _cgather(rout, p1f, p2f)
    out = _combine(xf.astype(_bf16), sw1.astype(_bf16), sw3.astype(_bf16),
                   sw2.astype(_bf16), o1, o2, s1, s2)
    return out.reshape(1, T, D)
